# baseline (speedup 1.0000x reference)
import jax
import jax.numpy as jnp
from jax.experimental import pallas as pl
from jax.experimental.pallas import tpu as pltpu

_TE = 1024
_TN = 128
_P = 2


def _silu(x):
    return x * jax.nn.sigmoid(x)


def _round_up(x, m):
    return ((x + m - 1) // m) * m


def _premul_kernel(nf_ref, wab_ref, a_ref, b_ref):
    x = nf_ref[...].astype(jnp.bfloat16)
    ab = jnp.dot(x, wab_ref[...], preferred_element_type=jnp.float32)
    h = a_ref.shape[1]
    a_ref[...] = ab[:, :h].astype(jnp.bfloat16)
    b_ref[...] = ab[:, h:]


def _latw_kernel(ips_ref, w_ref, o_ref):
    o_ref[...] = jnp.dot(ips_ref[...], w_ref[...],
                         precision=jax.lax.Precision.HIGHEST,
                         preferred_element_type=jnp.float32
                         ).astype(jnp.bfloat16)


def _fused_kernel(nt_ref, et_ref, fr_ref, la_ref, ev_ref,
                  nfa_ref, nf_ref, frac_ref, tg_ref, sid_ref, e2g_ref,
                  latw_ref, w1f_ref, eb1_ref, ew2_ref, eb2_ref,
                  nw1a_ref, nw1b_ref, nb1_ref, nw2_ref, nb2_ref,
                  o_ref, acc_ref, cnt_ref):
    c = pl.program_id(0)
    s = pl.program_id(1)
    tn = acc_ref.shape[0]
    te = tg_ref.shape[0]
    G = latw_ref.shape[0]
    H = acc_ref.shape[1]

    @pl.when(fr_ref[c, s] == 1)
    def _():
        acc_ref[...] = jnp.zeros_like(acc_ref)
        cnt_ref[...] = jnp.zeros_like(cnt_ref)

    @pl.when(ev_ref[c, s] == 1)
    def _():
        base = nt_ref[c, s] * tn
        ids = jax.lax.broadcasted_iota(jnp.int32, (tn, te), 0)
        msk = ids == sid_ref[...] - base
        mb = msk.astype(jnp.bfloat16)
        mf = msk.astype(jnp.float32)
        hi_pre = jax.lax.dot_general(
            mb, nfa_ref[...], (((0,), (0,)), ((), ())),
            preferred_element_type=jnp.float32)
        frac_sel = jax.lax.dot_general(
            mf, frac_ref[...], (((0,), (0,)), ((), ())),
            preferred_element_type=jnp.float32)
        frac_src = frac_sel[:, :3] + frac_sel[:, 3:]
        tg = tg_ref[...]
        frac_diff = jnp.mod(tg[:, H:H + 3] - frac_src, 1.0)
        gio = jax.lax.broadcasted_iota(jnp.int32, (G, te), 0)
        mg = (gio == e2g_ref[...]).astype(jnp.bfloat16)
        lat_c = jax.lax.dot_general(
            mg, latw_ref[...], (((0,), (0,)), ((), ())),
            preferred_element_type=jnp.float32)
        pre = (hi_pre + tg[:, :H] + lat_c
               + jnp.dot(frac_diff.astype(jnp.bfloat16), w1f_ref[...],
                         preferred_element_type=jnp.float32)
               + eb1_ref[...])
        h = _silu(pre).astype(jnp.bfloat16)
        ef = jnp.dot(h, ew2_ref[...], preferred_element_type=jnp.float32)
        ef = _silu(ef + eb2_ref[...]).astype(jnp.bfloat16)
        acc_ref[...] += jnp.dot(mb, ef, preferred_element_type=jnp.float32)
        cnt_ref[...] += jnp.sum(mf, axis=1, keepdims=True)

    @pl.when(la_ref[c, s] == 1)
    def _():
        inv = pl.reciprocal(jnp.maximum(cnt_ref[...], 1.0), approx=False)
        mean = acc_ref[...] * inv
        hn = (jnp.dot(nf_ref[...].astype(jnp.bfloat16), nw1a_ref[...],
                      preferred_element_type=jnp.float32)
              + jnp.dot(mean.astype(jnp.bfloat16), nw1b_ref[...],
                        preferred_element_type=jnp.float32)
              + nb1_ref[...])
        hn = _silu(hn).astype(jnp.bfloat16)
        h2 = jnp.dot(hn, nw2_ref[...], preferred_element_type=jnp.float32)
        o_ref[...] = nf_ref[...] + _silu(h2 + nb2_ref[...])


def kernel(node_features, frac_coords, lattices, edge_index, edge2graph,
           edge_w1_full, edge_w1_hihj, edge_w1_lf, edge_b1, edge_w2, edge_b2,
           node_w1_full, node_w1a, node_w1b, node_b1, node_w2, node_b2):
    N, H = node_features.shape
    E = edge_index.shape[1]
    G = lattices.shape[0]
    te, tn, P = _TE, _TN, _P

    E_pad = _round_up(E, te)
    N_pad = _round_up(N, tn * P)
    NE_T = E_pad // te
    NN_T = N_pad // tn
    TPC = NN_T // P
    CAP = NE_T + 2 * TPC + 2

    src = edge_index[0].astype(jnp.int32)
    dst = edge_index[1].astype(jnp.int32)
    e2g = edge2graph.astype(jnp.int32)
    if N * G < 2 ** 31:
        packed = dst * G + e2g
        src_s, packed_s = jax.lax.sort((src, packed), num_keys=1)
        dst_s = packed_s // G
        e2g_s = packed_s - dst_s * G
    else:
        src_s, dst_s, e2g_s = jax.lax.sort((src, dst, e2g), num_keys=1)

    if E_pad != E:
        padn = E_pad - E
        src_sp = jnp.concatenate([src_s, jnp.full((padn,), src_s[-1], jnp.int32)])
        src_row = jnp.concatenate([src_s, jnp.full((padn,), N_pad, jnp.int32)])
        dst_g = jnp.concatenate([dst_s, jnp.zeros((padn,), jnp.int32)])
        e2g_g = jnp.concatenate([e2g_s, jnp.zeros((padn,), jnp.int32)])
    else:
        src_sp = src_row = src_s
        dst_g = dst_s
        e2g_g = e2g_s
    src_row = src_row.reshape(1, E_pad)
    e2g_row = e2g_g.reshape(1, E_pad)

    nf_p = node_features if N_pad == N else jnp.concatenate(
        [node_features, jnp.zeros((N_pad - N, H), node_features.dtype)], axis=0)
    frac_p = frac_coords if N_pad == N else jnp.concatenate(
        [frac_coords, jnp.zeros((N_pad - N, 3), frac_coords.dtype)], axis=0)
    frac_hi = jnp.floor(frac_p * 64.0) * (1.0 / 64.0)
    frac6_p = jnp.concatenate([frac_hi, frac_p - frac_hi], axis=1)

    wab = jnp.concatenate([edge_w1_hihj[:H], edge_w1_hihj[H:]],
                          axis=1).astype(jnp.bfloat16)
    BN = 2048 if N_pad % 2048 == 0 else tn
    nfa, nfb = pl.pallas_call(
        _premul_kernel,
        out_shape=(jax.ShapeDtypeStruct((N_pad, H), jnp.bfloat16),
                   jax.ShapeDtypeStruct((N_pad, H), jnp.float32)),
        grid=(N_pad // BN,),
        in_specs=[pl.BlockSpec((BN, H), lambda i: (i, 0)),
                  pl.BlockSpec((H, 2 * H), lambda i: (0, 0))],
        out_specs=(pl.BlockSpec((BN, H), lambda i: (i, 0)),
                   pl.BlockSpec((BN, H), lambda i: (i, 0))),
        compiler_params=pltpu.CompilerParams(
            dimension_semantics=("parallel",)),
    )(nf_p, wab)

    lat_ips = jnp.einsum('gij,gkj->gik', lattices, lattices).reshape(G, 9)
    latw = pl.pallas_call(
        _latw_kernel,
        out_shape=jax.ShapeDtypeStruct((G, H), jnp.bfloat16),
        in_specs=[pl.BlockSpec((G, 9), lambda: (0, 0)),
                  pl.BlockSpec((9, H), lambda: (0, 0))],
        out_specs=pl.BlockSpec((G, H), lambda: (0, 0)),
    )(lat_ips, edge_w1_lf[:9])

    tdst = jnp.concatenate([nfb, frac_p], axis=1)
    tg = tdst[dst_g]

    src_tiles = src_sp.reshape(NE_T, te)
    a = src_tiles[:, 0] // tn
    b = src_tiles[:, -1] // tn
    ii = jnp.arange(NN_T, dtype=jnp.int32)
    k_lo = jnp.searchsorted(b, ii, side='left').astype(jnp.int32)
    k_hi = (jnp.searchsorted(a, ii, side='right') - 1).astype(jnp.int32)
    ov = jnp.maximum(k_hi - k_lo + 1, 0)

    ov_c = ov.reshape(P, TPC)
    steps = jnp.maximum(ov_c, 1)
    off = jnp.cumsum(steps, axis=1) - steps
    total = off[:, -1] + steps[:, -1]
    parr = jnp.arange(CAP, dtype=jnp.int32)
    tloc = jnp.sum(off[:, :, None] <= parr[None, None, :], axis=1) - 1
    tloc = jnp.clip(tloc, 0, TPC - 1)
    valid = parr[None, :] < total[:, None]
    offp = jnp.take_along_axis(off, tloc, axis=1)
    j = parr[None, :] - offp
    nt = tloc + (jnp.arange(P, dtype=jnp.int32) * TPC)[:, None]
    ovp = jnp.take_along_axis(ov_c, tloc, axis=1)
    stp = jnp.maximum(ovp, 1)
    ev = valid & (ovp > 0)
    et_raw = jnp.where(ev, k_lo[nt] + j, -1)
    fr = (valid & (j == 0)).astype(jnp.int32)
    la = (valid & (j == stp - 1)).astype(jnp.int32)
    et_dma = jnp.maximum(jax.lax.cummax(et_raw, axis=1), 0).astype(jnp.int32)
    nt_map = nt.astype(jnp.int32)
    ev = ev.astype(jnp.int32)

    w1f = edge_w1_lf[9:12].astype(jnp.bfloat16)
    ew2 = edge_w2.astype(jnp.bfloat16)
    nw1a = node_w1a.astype(jnp.bfloat16)
    nw1b = node_w1b.astype(jnp.bfloat16)
    nw2 = node_w2.astype(jnp.bfloat16)

    def nt_ix(c, s, nt_r, et_r, fr_r, la_r, ev_r):
        return (nt_r[c, s], 0)

    def et_ix(c, s, nt_r, et_r, fr_r, la_r, ev_r):
        return (et_r[c, s], 0)

    def row_ix(c, s, nt_r, et_r, fr_r, la_r, ev_r):
        return (0, et_r[c, s])

    def w_ix(c, s, nt_r, et_r, fr_r, la_r, ev_r):
        return (0, 0)

    out = pl.pallas_call(
        _fused_kernel,
        out_shape=jax.ShapeDtypeStruct((N_pad, H), jnp.float32),
        grid_spec=pltpu.PrefetchScalarGridSpec(
            num_scalar_prefetch=5,
            grid=(P, CAP),
            in_specs=[
                pl.BlockSpec((tn, H), nt_ix),
                pl.BlockSpec((tn, H), nt_ix),
                pl.BlockSpec((tn, 6), nt_ix),
                pl.BlockSpec((te, H + 3), et_ix),
                pl.BlockSpec((1, te), row_ix),
                pl.BlockSpec((1, te), row_ix),
                pl.BlockSpec((G, H), w_ix),
                pl.BlockSpec((3, H), w_ix),
                pl.BlockSpec((1, H), w_ix),
                pl.BlockSpec((H, H), w_ix),
                pl.BlockSpec((1, H), w_ix),
                pl.BlockSpec((H, H), w_ix),
                pl.BlockSpec((H, H), w_ix),
                pl.BlockSpec((1, H), w_ix),
                pl.BlockSpec((H, H), w_ix),
                pl.BlockSpec((1, H), w_ix),
            ],
            out_specs=pl.BlockSpec((tn, H), nt_ix),
            scratch_shapes=[pltpu.VMEM((tn, H), jnp.float32),
                            pltpu.VMEM((tn, 1), jnp.float32)]),
        compiler_params=pltpu.CompilerParams(
            dimension_semantics=("parallel", "arbitrary"),
            vmem_limit_bytes=64 * 1024 * 1024),
    )(nt_map, et_dma, fr, la, ev,
      nfa, nf_p, frac6_p, tg, src_row, e2g_row,
      latw, w1f, edge_b1, ew2, edge_b2,
      nw1a, nw1b, node_b1, nw2, node_b2)

    return out[:N]


# speedup vs baseline: 1.0691x; 1.0691x over previous
import jax
import jax.numpy as jnp
from jax.experimental import pallas as pl
from jax.experimental.pallas import tpu as pltpu

_TE = 1024
_TN = 256
_P = 2


def _silu(x):
    return x * jax.nn.sigmoid(x)


def _round_up(x, m):
    return ((x + m - 1) // m) * m


def _premul_kernel(nf_ref, wab_ref, a_ref, b_ref):
    x = nf_ref[...].astype(jnp.bfloat16)
    ab = jnp.dot(x, wab_ref[...], preferred_element_type=jnp.float32)
    h = a_ref.shape[1]
    a_ref[...] = ab[:, :h].astype(jnp.bfloat16)
    b_ref[...] = ab[:, h:]


def _latw_kernel(ips_ref, w_ref, o_ref):
    o_ref[...] = jnp.dot(ips_ref[...], w_ref[...],
                         precision=jax.lax.Precision.HIGHEST,
                         preferred_element_type=jnp.float32
                         ).astype(jnp.bfloat16)


def _fused_kernel(nt_ref, et_ref, fr_ref, la_ref, ev_ref,
                  nfa_ref, nf_ref, frac_ref, tg_ref, sid_ref, e2g_ref,
                  latw_ref, w1f_ref, eb1_ref, ew2_ref, eb2_ref,
                  nw1a_ref, nw1b_ref, nb1_ref, nw2_ref, nb2_ref,
                  o_ref, acc_ref, cnt_ref):
    c = pl.program_id(0)
    s = pl.program_id(1)
    tn = acc_ref.shape[0]
    te = tg_ref.shape[0]
    G = latw_ref.shape[0]
    H = acc_ref.shape[1]

    @pl.when(fr_ref[c, s] == 1)
    def _():
        acc_ref[...] = jnp.zeros_like(acc_ref)
        cnt_ref[...] = jnp.zeros_like(cnt_ref)

    @pl.when(ev_ref[c, s] == 1)
    def _():
        base = nt_ref[c, s] * tn
        ids = jax.lax.broadcasted_iota(jnp.int32, (tn, te), 0)
        msk = ids == sid_ref[...] - base
        mb = msk.astype(jnp.bfloat16)
        mf = msk.astype(jnp.float32)
        hi_pre = jax.lax.dot_general(
            mb, nfa_ref[...], (((0,), (0,)), ((), ())),
            preferred_element_type=jnp.float32)
        frac_sel = jax.lax.dot_general(
            mf, frac_ref[...], (((0,), (0,)), ((), ())),
            preferred_element_type=jnp.float32)
        frac_src = frac_sel[:, :3] + frac_sel[:, 3:]
        tg = tg_ref[...]
        frac_diff = jnp.mod(tg[:, H:H + 3] - frac_src, 1.0)
        gio = jax.lax.broadcasted_iota(jnp.int32, (G, te), 0)
        mg = (gio == e2g_ref[...]).astype(jnp.bfloat16)
        lat_c = jax.lax.dot_general(
            mg, latw_ref[...], (((0,), (0,)), ((), ())),
            preferred_element_type=jnp.float32)
        pre = (hi_pre + tg[:, :H] + lat_c
               + jnp.dot(frac_diff.astype(jnp.bfloat16), w1f_ref[...],
                         preferred_element_type=jnp.float32)
               + eb1_ref[...])
        h = _silu(pre).astype(jnp.bfloat16)
        ef = jnp.dot(h, ew2_ref[...], preferred_element_type=jnp.float32)
        ef = _silu(ef + eb2_ref[...]).astype(jnp.bfloat16)
        acc_ref[...] += jnp.dot(mb, ef, preferred_element_type=jnp.float32)
        cnt_ref[...] += jnp.sum(mf, axis=1, keepdims=True)

    @pl.when(la_ref[c, s] == 1)
    def _():
        inv = pl.reciprocal(jnp.maximum(cnt_ref[...], 1.0), approx=False)
        mean = acc_ref[...] * inv
        hn = (jnp.dot(nf_ref[...].astype(jnp.bfloat16), nw1a_ref[...],
                      preferred_element_type=jnp.float32)
              + jnp.dot(mean.astype(jnp.bfloat16), nw1b_ref[...],
                        preferred_element_type=jnp.float32)
              + nb1_ref[...])
        hn = _silu(hn).astype(jnp.bfloat16)
        h2 = jnp.dot(hn, nw2_ref[...], preferred_element_type=jnp.float32)
        o_ref[...] = nf_ref[...] + _silu(h2 + nb2_ref[...])


def kernel(node_features, frac_coords, lattices, edge_index, edge2graph,
           edge_w1_full, edge_w1_hihj, edge_w1_lf, edge_b1, edge_w2, edge_b2,
           node_w1_full, node_w1a, node_w1b, node_b1, node_w2, node_b2):
    N, H = node_features.shape
    E = edge_index.shape[1]
    G = lattices.shape[0]
    te, tn, P = _TE, _TN, _P

    E_pad = _round_up(E, te)
    N_pad = _round_up(N, tn * P)
    NE_T = E_pad // te
    NN_T = N_pad // tn
    TPC = NN_T // P
    CAP = NE_T + 2 * TPC + 2

    src = edge_index[0].astype(jnp.int32)
    dst = edge_index[1].astype(jnp.int32)
    e2g = edge2graph.astype(jnp.int32)
    if N * G < 2 ** 31:
        packed = dst * G + e2g
        src_s, packed_s = jax.lax.sort((src, packed), num_keys=1)
        dst_s = packed_s // G
        e2g_s = packed_s - dst_s * G
    else:
        src_s, dst_s, e2g_s = jax.lax.sort((src, dst, e2g), num_keys=1)

    if E_pad != E:
        padn = E_pad - E
        src_sp = jnp.concatenate([src_s, jnp.full((padn,), src_s[-1], jnp.int32)])
        src_row = jnp.concatenate([src_s, jnp.full((padn,), N_pad, jnp.int32)])
        dst_g = jnp.concatenate([dst_s, jnp.zeros((padn,), jnp.int32)])
        e2g_g = jnp.concatenate([e2g_s, jnp.zeros((padn,), jnp.int32)])
    else:
        src_sp = src_row = src_s
        dst_g = dst_s
        e2g_g = e2g_s
    src_row = src_row.reshape(1, E_pad)
    e2g_row = e2g_g.reshape(1, E_pad)

    nf_p = node_features if N_pad == N else jnp.concatenate(
        [node_features, jnp.zeros((N_pad - N, H), node_features.dtype)], axis=0)
    frac_p = frac_coords if N_pad == N else jnp.concatenate(
        [frac_coords, jnp.zeros((N_pad - N, 3), frac_coords.dtype)], axis=0)
    frac_hi = jnp.floor(frac_p * 64.0) * (1.0 / 64.0)
    frac6_p = jnp.concatenate([frac_hi, frac_p - frac_hi], axis=1)

    wab = jnp.concatenate([edge_w1_hihj[:H], edge_w1_hihj[H:]],
                          axis=1).astype(jnp.bfloat16)
    BN = 2048 if N_pad % 2048 == 0 else tn
    nfa, nfb = pl.pallas_call(
        _premul_kernel,
        out_shape=(jax.ShapeDtypeStruct((N_pad, H), jnp.bfloat16),
                   jax.ShapeDtypeStruct((N_pad, H), jnp.float32)),
        grid=(N_pad // BN,),
        in_specs=[pl.BlockSpec((BN, H), lambda i: (i, 0)),
                  pl.BlockSpec((H, 2 * H), lambda i: (0, 0))],
        out_specs=(pl.BlockSpec((BN, H), lambda i: (i, 0)),
                   pl.BlockSpec((BN, H), lambda i: (i, 0))),
        compiler_params=pltpu.CompilerParams(
            dimension_semantics=("parallel",)),
    )(nf_p, wab)

    lat_ips = jnp.einsum('gij,gkj->gik', lattices, lattices).reshape(G, 9)
    latw = pl.pallas_call(
        _latw_kernel,
        out_shape=jax.ShapeDtypeStruct((G, H), jnp.bfloat16),
        in_specs=[pl.BlockSpec((G, 9), lambda: (0, 0)),
                  pl.BlockSpec((9, H), lambda: (0, 0))],
        out_specs=pl.BlockSpec((G, H), lambda: (0, 0)),
    )(lat_ips, edge_w1_lf[:9])

    tdst = jnp.concatenate([nfb, frac_p], axis=1)
    tg = tdst[dst_g]

    src_tiles = src_sp.reshape(NE_T, te)
    a = src_tiles[:, 0] // tn
    b = src_tiles[:, -1] // tn
    ii = jnp.arange(NN_T, dtype=jnp.int32)
    k_lo = jnp.searchsorted(b, ii, side='left').astype(jnp.int32)
    k_hi = (jnp.searchsorted(a, ii, side='right') - 1).astype(jnp.int32)
    ov = jnp.maximum(k_hi - k_lo + 1, 0)

    ov_c = ov.reshape(P, TPC)
    steps = jnp.maximum(ov_c, 1)
    off = jnp.cumsum(steps, axis=1) - steps
    total = off[:, -1] + steps[:, -1]
    parr = jnp.arange(CAP, dtype=jnp.int32)
    tloc = jnp.sum(off[:, :, None] <= parr[None, None, :], axis=1) - 1
    tloc = jnp.clip(tloc, 0, TPC - 1)
    valid = parr[None, :] < total[:, None]
    offp = jnp.take_along_axis(off, tloc, axis=1)
    j = parr[None, :] - offp
    nt = tloc + (jnp.arange(P, dtype=jnp.int32) * TPC)[:, None]
    ovp = jnp.take_along_axis(ov_c, tloc, axis=1)
    stp = jnp.maximum(ovp, 1)
    ev = valid & (ovp > 0)
    et_raw = jnp.where(ev, k_lo[nt] + j, -1)
    fr = (valid & (j == 0)).astype(jnp.int32)
    la = (valid & (j == stp - 1)).astype(jnp.int32)
    et_dma = jnp.maximum(jax.lax.cummax(et_raw, axis=1), 0).astype(jnp.int32)
    nt_map = nt.astype(jnp.int32)
    ev = ev.astype(jnp.int32)

    w1f = edge_w1_lf[9:12].astype(jnp.bfloat16)
    ew2 = edge_w2.astype(jnp.bfloat16)
    nw1a = node_w1a.astype(jnp.bfloat16)
    nw1b = node_w1b.astype(jnp.bfloat16)
    nw2 = node_w2.astype(jnp.bfloat16)

    def nt_ix(c, s, nt_r, et_r, fr_r, la_r, ev_r):
        return (nt_r[c, s], 0)

    def et_ix(c, s, nt_r, et_r, fr_r, la_r, ev_r):
        return (et_r[c, s], 0)

    def row_ix(c, s, nt_r, et_r, fr_r, la_r, ev_r):
        return (0, et_r[c, s])

    def w_ix(c, s, nt_r, et_r, fr_r, la_r, ev_r):
        return (0, 0)

    out = pl.pallas_call(
        _fused_kernel,
        out_shape=jax.ShapeDtypeStruct((N_pad, H), jnp.float32),
        grid_spec=pltpu.PrefetchScalarGridSpec(
            num_scalar_prefetch=5,
            grid=(P, CAP),
            in_specs=[
                pl.BlockSpec((tn, H), nt_ix),
                pl.BlockSpec((tn, H), nt_ix),
                pl.BlockSpec((tn, 6), nt_ix),
                pl.BlockSpec((te, H + 3), et_ix),
                pl.BlockSpec((1, te), row_ix),
                pl.BlockSpec((1, te), row_ix),
                pl.BlockSpec((G, H), w_ix),
                pl.BlockSpec((3, H), w_ix),
                pl.BlockSpec((1, H), w_ix),
                pl.BlockSpec((H, H), w_ix),
                pl.BlockSpec((1, H), w_ix),
                pl.BlockSpec((H, H), w_ix),
                pl.BlockSpec((H, H), w_ix),
                pl.BlockSpec((1, H), w_ix),
                pl.BlockSpec((H, H), w_ix),
                pl.BlockSpec((1, H), w_ix),
            ],
            out_specs=pl.BlockSpec((tn, H), nt_ix),
            scratch_shapes=[pltpu.VMEM((tn, H), jnp.float32),
                            pltpu.VMEM((tn, 1), jnp.float32)]),
        compiler_params=pltpu.CompilerParams(
            dimension_semantics=("parallel", "arbitrary"),
            vmem_limit_bytes=64 * 1024 * 1024),
    )(nt_map, et_dma, fr, la, ev,
      nfa, nf_p, frac6_p, tg, src_row, e2g_row,
      latw, w1f, edge_b1, ew2, edge_b2,
      nw1a, nw1b, node_b1, nw2, node_b2)

    return out[:N]
